# baseline (speedup 1.0000x reference)
import functools

import jax
import jax.numpy as jnp
from jax import lax
from jax.experimental import pallas as pl
from jax.experimental.pallas import tpu as pltpu
from jax.experimental.pallas import tpu_sc as plsc

EMBED = 128
MODEL = 1024
OFF_O, OFF_D, OFF_V = 128, 1152, 1664
ROWS = 1792


def _fuse_body(pt_ref, ot_ref, dt_ref, vt_ref, w_ref, b_ref, f_ref):
    hp = jax.lax.Precision.HIGHEST
    b = b_ref[...]
    f_ref[0:128, :] = (
        jnp.dot(pt_ref[...], w_ref[0:128, :], precision=hp,
                preferred_element_type=jnp.float32) + b[None, :]
    )
    f_ref[128:1152, :] = jnp.dot(ot_ref[...], w_ref[128:256, :], precision=hp,
                                 preferred_element_type=jnp.float32)
    f_ref[1152:1664, :] = jnp.dot(dt_ref[...], w_ref[256:384, :], precision=hp,
                                  preferred_element_type=jnp.float32)
    f_ref[1664:1792, :] = jnp.dot(vt_ref[...], w_ref[384:512, :], precision=hp,
                                  preferred_element_type=jnp.float32)


def _make_sc_kernel(n_tok):
    info = plsc.get_sparse_core_info()
    nc, ns = info.num_cores, info.num_subcores
    nw = nc * ns
    n_w = n_tok // nw
    C = 16
    nch = n_w // C

    mesh = plsc.VectorSubcoreMesh(core_axis_name="c", subcore_axis_name="s")

    @functools.partial(
        pl.kernel,
        mesh=mesh,
        out_type=jax.ShapeDtypeStruct((n_tok, MODEL), jnp.float32),
        scratch_types=[
            pltpu.VMEM((n_w * 4,), jnp.int32),
            pltpu.VMEM((n_w * 4,), jnp.int32),
            pltpu.VMEM((C * 4, MODEL), jnp.float32),
            pltpu.VMEM((C, MODEL), jnp.float32),
            pltpu.VMEM((16,), jnp.int32),
            pltpu.SemaphoreType.DMA,
        ],
    )
    def sc_gather(f_hbm, idx_hbm, offs_hbm, out_hbm, idxr_v, idxf_v, buf_v,
                  outb_v, offs_v, sem):
        wid = lax.axis_index("s") * nc + lax.axis_index("c")
        base = wid * n_w
        pltpu.sync_copy(idx_hbm.at[pl.ds(base * 4, n_w * 4)], idxr_v)
        pltpu.sync_copy(offs_hbm, offs_v)
        offs = offs_v[...]

        def prep(i, carry):
            idxf_v[pl.ds(i * 16, 16)] = idxr_v[pl.ds(i * 16, 16)] + offs
            return carry

        lax.fori_loop(0, n_w * 4 // 16, prep, 0)

        def chunk_body(ch, carry):
            tok = ch * C
            pltpu.async_copy(
                f_hbm.at[idxf_v.at[pl.ds(tok * 4, C * 4)]], buf_v, sem
            ).wait()

            def col_body(j, inner):
                s = pl.ds(j * 16, 16)
                for t in range(C):
                    outb_v[t, s] = (
                        buf_v[4 * t, s] + buf_v[4 * t + 1, s]
                        + buf_v[4 * t + 2, s] + buf_v[4 * t + 3, s]
                    )
                return inner

            lax.fori_loop(0, MODEL // 16, col_body, 0)
            pltpu.sync_copy(outb_v, out_hbm.at[pl.ds(base + tok, C)])
            return carry

        lax.fori_loop(0, nch, chunk_body, 0)

    return sc_gather


def kernel(x, pitch_table, onset_table, duration_table, velocity_table, W, b):
    bsz, seq, _ = x.shape
    n_tok = bsz * seq

    fused = pl.pallas_call(
        _fuse_body,
        out_shape=jax.ShapeDtypeStruct((ROWS, MODEL), jnp.float32),
    )(pitch_table, onset_table, duration_table, velocity_table, W, b)

    idx_flat = x.reshape(-1).astype(jnp.int32)
    offs = jnp.array([0, OFF_O, OFF_D, OFF_V] * 4, dtype=jnp.int32)
    out = _make_sc_kernel(n_tok)(fused, idx_flat, offs)
    return out.reshape(bsz, seq, MODEL)


# speedup vs baseline: 1.3346x; 1.3346x over previous
import functools

import jax
import jax.numpy as jnp
from jax import lax
from jax.experimental import pallas as pl
from jax.experimental.pallas import tpu as pltpu
from jax.experimental.pallas import tpu_sc as plsc

EMBED = 128
MODEL = 1024
OFF_O, OFF_D, OFF_V = 128, 1152, 1664
ROWS = 1792


def _fuse_body(pt_ref, ot_ref, dt_ref, vt_ref, w_ref, b_ref, f_ref):
    hp = jax.lax.Precision.HIGHEST
    b = b_ref[...]
    f_ref[0:128, :] = (
        jnp.dot(pt_ref[...], w_ref[0:128, :], precision=hp,
                preferred_element_type=jnp.float32) + b[None, :]
    )
    f_ref[128:1152, :] = jnp.dot(ot_ref[...], w_ref[128:256, :], precision=hp,
                                 preferred_element_type=jnp.float32)
    f_ref[1152:1664, :] = jnp.dot(dt_ref[...], w_ref[256:384, :], precision=hp,
                                  preferred_element_type=jnp.float32)
    f_ref[1664:1792, :] = jnp.dot(vt_ref[...], w_ref[384:512, :], precision=hp,
                                  preferred_element_type=jnp.float32)


def _make_sc_kernel(n_tok):
    info = plsc.get_sparse_core_info()
    nc, ns = info.num_cores, info.num_subcores
    nw = nc * ns
    n_w = n_tok // nw
    C = 8
    nch = n_w // C

    mesh = plsc.VectorSubcoreMesh(core_axis_name="c", subcore_axis_name="s")

    @functools.partial(
        pl.kernel,
        mesh=mesh,
        out_type=jax.ShapeDtypeStruct((n_tok, MODEL), jnp.float32),
        scratch_types=[
            pltpu.VMEM((n_w * 4,), jnp.int32),
            pltpu.VMEM((n_w * 4,), jnp.int32),
            pltpu.VMEM((C * 4, MODEL), jnp.float32),
            pltpu.VMEM((C * 4, MODEL), jnp.float32),
            pltpu.VMEM((C, MODEL), jnp.float32),
            pltpu.VMEM((C, MODEL), jnp.float32),
            pltpu.VMEM((16,), jnp.int32),
            pltpu.SemaphoreType.DMA,
            pltpu.SemaphoreType.DMA,
            pltpu.SemaphoreType.DMA,
            pltpu.SemaphoreType.DMA,
        ],
    )
    def sc_gather(f_hbm, idx_hbm, offs_hbm, out_hbm, idxr_v, idxf_v, buf0_v,
                  buf1_v, outb0_v, outb1_v, offs_v, gsem0, gsem1, wsem0,
                  wsem1):
        wid = lax.axis_index("s") * nc + lax.axis_index("c")
        base = wid * n_w
        pltpu.sync_copy(idx_hbm.at[pl.ds(base * 4, n_w * 4)], idxr_v)
        pltpu.sync_copy(offs_hbm, offs_v)
        offs = offs_v[...]

        def prep(i, carry):
            idxf_v[pl.ds(i * 16, 16)] = idxr_v[pl.ds(i * 16, 16)] + offs
            return carry

        lax.fori_loop(0, n_w * 4 // 16, prep, 0)

        bufs = (buf0_v, buf1_v)
        outbs = (outb0_v, outb1_v)
        gsems = (gsem0, gsem1)
        wsems = (wsem0, wsem1)

        def gather_start(ch, b):
            pltpu.async_copy(
                f_hbm.at[idxf_v.at[pl.ds(ch * C * 4, C * 4)]], bufs[b],
                gsems[b])

        gather_start(0, 0)

        def chunk_pair(ch2, carry):
            ch0 = ch2 * 2
            for b in range(2):
                ch = ch0 + b
                pltpu.make_async_copy(
                    f_hbm.at[idxf_v.at[pl.ds(0, C * 4)]], bufs[b],
                    gsems[b]).wait()

                @pl.when(ch + 1 < nch)
                def _():
                    gather_start(ch + 1, 1 - b)

                @pl.when(ch >= 2)
                def _():
                    pltpu.make_async_copy(
                        outbs[b], out_hbm.at[pl.ds(base, C)], wsems[b]).wait()

                buf_v = bufs[b]
                outb_v = outbs[b]

                def col_body(j, inner):
                    s = pl.ds(j * 16, 16)
                    for t in range(C):
                        outb_v[t, s] = (
                            buf_v[4 * t, s] + buf_v[4 * t + 1, s]
                            + buf_v[4 * t + 2, s] + buf_v[4 * t + 3, s]
                        )
                    return inner

                lax.fori_loop(0, MODEL // 16, col_body, 0)
                pltpu.async_copy(
                    outb_v, out_hbm.at[pl.ds(base + ch * C, C)], wsems[b])
            return carry

        lax.fori_loop(0, nch // 2, chunk_pair, 0)
        for b in range(2):
            pltpu.make_async_copy(
                outbs[b], out_hbm.at[pl.ds(base, C)], wsems[b]).wait()

    return sc_gather


def kernel(x, pitch_table, onset_table, duration_table, velocity_table, W, b):
    bsz, seq, _ = x.shape
    n_tok = bsz * seq

    fused = pl.pallas_call(
        _fuse_body,
        out_shape=jax.ShapeDtypeStruct((ROWS, MODEL), jnp.float32),
    )(pitch_table, onset_table, duration_table, velocity_table, W, b)

    idx_flat = x.reshape(-1).astype(jnp.int32)
    offs = jnp.array([0, OFF_O, OFF_D, OFF_V] * 4, dtype=jnp.int32)
    out = _make_sc_kernel(n_tok)(fused, idx_flat, offs)
    return out.reshape(bsz, seq, MODEL)
